# baseline (speedup 1.0000x reference)
import dataclasses
import functools

import jax
import jax.numpy as jnp
from jax import lax
from jax.experimental import pallas as pl
from jax.experimental.pallas import tpu as pltpu
from jax.experimental.pallas import tpu_sc as plsc

_N = 10000
_G = 64
_E = 320000
_C = 2
_NCORES = 2
_NSUB = 16
_NW = _NCORES * _NSUB
_BLK = 128
_NBLK = 84
_EW = _BLK * _NBLK
_EPAD = _EW * _NW
_NPAD = 10240
_ROWS_OUT = _NPAD // _NSUB


def _sc_compiler_params():
    cp = pltpu.CompilerParams()
    fields = pltpu.CompilerParams.__dataclass_fields__
    if "needs_layout_passes" in fields:
        cp = dataclasses.replace(cp, needs_layout_passes=False)
    if "use_tc_tiling_on_sc" in fields:
        cp = dataclasses.replace(cp, use_tc_tiling_on_sc=False)
    return cp


@functools.lru_cache(maxsize=None)
def _seg_scatter(d: int):
    ld = d.bit_length() - 1
    mesh = plsc.VectorSubcoreMesh(core_axis_name="c", subcore_axis_name="s")

    @functools.partial(
        pl.kernel,
        out_type=jax.ShapeDtypeStruct((_NCORES, _NPAD, d), jnp.float32),
        mesh=mesh,
        compiler_params=_sc_compiler_params(),
        scratch_types=[
            pltpu.VMEM((_NBLK, _BLK), jnp.int32),
            pltpu.VMEM((_NBLK, _BLK), jnp.int32),
            pltpu.VMEM((_NBLK, _BLK), jnp.float32),
            pltpu.VMEM((_BLK, d), jnp.float32),
            pltpu.VMEM((_BLK, d), jnp.float32),
            pltpu.VMEM((_BLK, d), jnp.float32),
            pltpu.VMEM((_BLK, d), jnp.float32),
            pltpu.VMEM_SHARED((_NPAD, d), jnp.float32),
            pltpu.SemaphoreType.DMA,
            pltpu.SemaphoreType.DMA,
            pltpu.SemaphoreType.DMA,
            pltpu.SemaphoreType.DMA,
            pltpu.SemaphoreType.DMA,
            pltpu.SemaphoreType.DMA,
            pltpu.SemaphoreType.DMA,
            pltpu.SemaphoreType.DMA,
        ],
    )
    def k(h_hbm, src_hbm, dst_hbm, w_hbm, out_hbm,
          src_v, dst_v, w_v, r0, r1, r2, r3, agg_sh,
          g0, g1, g2, g3, s0, s1, s2, s3):
        c = lax.axis_index("c")
        s = lax.axis_index("s")
        wid = c * _NSUB + s
        iota = lax.iota(jnp.int32, 16)
        zeros = jnp.zeros((16,), jnp.float32)
        rows = [r0, r1, r2, r3]
        gsem = [g0, g1, g2, g3]
        ssem = [s0, s1, s2, s3]

        def g_issue(j, t):
            pltpu.async_copy(h_hbm.at[src_v.at[j]], rows[t], gsem[t])

        def g_wait(j, t):
            pltpu.make_async_copy(h_hbm.at[src_v.at[j]], rows[t],
                                  gsem[t]).wait()

        def s_issue(j, t):
            pltpu.async_copy(rows[t], agg_sh.at[dst_v.at[j]], ssem[t],
                             add=True)

        def s_wait(j, t):
            pltpu.make_async_copy(rows[t], agg_sh.at[dst_v.at[j]],
                                  ssem[t]).wait()

        def mul(t, j):
            j_splat = lax.full((16,), j, jnp.int32)
            r = rows[t]
            if d >= 16:
                unr = 4
                @pl.loop(0, _BLK, step=unr)
                def _(e0):
                    for u in range(unr):
                        e = e0 + u
                        wv = plsc.load_gather(
                            w_v, [j_splat, lax.full((16,), e, jnp.int32)])
                        for kk in range(d // 16):
                            rv = r[e, pl.ds(kk * 16, 16)]
                            r[e, pl.ds(kk * 16, 16)] = rv * wv
            else:
                @pl.loop(0, _BLK * d, step=64)
                def _(p0):
                    for u in range(4):
                        v = p0 + u * 16 + iota
                        ee = v >> ld
                        cc = v & (d - 1)
                        wv = plsc.load_gather(w_v, [j_splat, ee])
                        rv = plsc.load_gather(r, [ee, cc])
                        plsc.store_scatter(r, [ee, cc], rv * wv)

        pltpu.sync_copy(src_hbm.at[wid], src_v)
        pltpu.sync_copy(dst_hbm.at[wid], dst_v)
        pltpu.sync_copy(w_hbm.at[wid], w_v)
        g_issue(0, 0)
        g_issue(1, 1)

        @pl.loop(0, _BLK * d, step=16)
        def _(p):
            v = p + iota
            plsc.store_scatter(r2, [v >> ld, v & (d - 1)], zeros)

        @pl.loop(0, _ROWS_OUT // _BLK)
        def _(i):
            pltpu.sync_copy(r2,
                            agg_sh.at[pl.ds(s * _ROWS_OUT + i * _BLK, _BLK)])

        plsc.subcore_barrier()

        @pl.loop(0, _NBLK, step=4)
        def _(jj):
            for t in range(4):
                j = jj + t
                tn = (t + 2) % 4
                g_wait(j, t)
                mul(t, j)
                s_issue(j, t)

                @pl.when(j + 2 < _NBLK)
                def _():
                    if t < 2:
                        @pl.when(jj > 0)
                        def _():
                            s_wait(j - 2, tn)
                    else:
                        s_wait(j - 2, tn)
                    g_issue(j + 2, tn)

        for t in range(4):
            s_wait(_NBLK - 4 + t, t)
        plsc.subcore_barrier()
        pltpu.sync_copy(agg_sh.at[pl.ds(s * _ROWS_OUT, _ROWS_OUT)],
                        out_hbm.at[c, pl.ds(s * _ROWS_OUT, _ROWS_OUT)])

    return k


def _dot(a, b):
    return jnp.dot(a, b, preferred_element_type=jnp.float32)


def _bi_body(a_ref, h_ref, wr_ref, br_ref, wt_ref, o_ref):
    agg = a_ref[0, :_N, :] + a_ref[1, :_N, :]
    o_ref[...] = (_dot(agg, wr_ref[...]) + br_ref[...]
                  + _dot(h_ref[...], wt_ref[...]))


def _b1_body(a0_ref, a1_ref, h_ref, wr_ref, br_ref, wt_ref, o_ref):
    agg = jnp.concatenate(
        [a0_ref[0, :_N, :] + a0_ref[1, :_N, :],
         a1_ref[0, :_N, :] + a1_ref[1, :_N, :]], axis=1)
    o_ref[...] = (_dot(agg, wr_ref[...]) + br_ref[...]
                  + _dot(h_ref[...], wt_ref[...]))


def _fin_body(a_ref, h_ref, wr_ref, br_ref, wt_ref, batch_ref,
              wl_ref, bl_ref, o_ref):
    agg = a_ref[0, :_N, :] + a_ref[1, :_N, :]
    h5 = (_dot(agg, wr_ref[...]) + br_ref[...]
          + _dot(h_ref[...], wt_ref[...]))
    sel = (batch_ref[...] == lax.broadcasted_iota(jnp.int32, (_G, _N), 0))
    pooled = jnp.dot(sel.astype(jnp.float32), h5,
                     preferred_element_type=jnp.float32,
                     precision=lax.Precision.HIGHEST)
    o_ref[...] = _dot(pooled, wl_ref[...]) + bl_ref[...]


def _tc(body, out_shape, *args):
    return pl.pallas_call(
        body, out_shape=jax.ShapeDtypeStruct(out_shape, jnp.float32))(*args)


def kernel(x, edge_index, edge_attr, batch,
           W_rel1, b_rel1, W_root1,
           W_rel2, b_rel2, W_root2,
           W_rel3, b_rel3, W_root3,
           W_rel4, b_rel4, W_root4,
           W_rel5, b_rel5, W_root5,
           W_lin, b_lin):
    f32 = jnp.float32
    src = edge_index[0]
    dst = edge_index[1]
    ppw = _EW - _E // _NW
    pad_idx = (jnp.arange(_NW, dtype=jnp.int32)[:, None] * ppw
               + jnp.arange(ppw, dtype=jnp.int32)[None, :]) % _N
    srcp = jnp.concatenate([src.reshape(_NW, _E // _NW), pad_idx],
                           axis=1).reshape(_NW, _NBLK, _BLK)
    dstp = jnp.concatenate([dst.reshape(_NW, _E // _NW), pad_idx],
                           axis=1).reshape(_NW, _NBLK, _BLK)
    wp = jnp.concatenate([edge_attr.reshape(_NW, _E // _NW),
                          jnp.zeros((_NW, ppw), f32)],
                         axis=1).reshape(_NW, _NBLK, _BLK)
    batch2 = batch.reshape(1, _N)

    agg1a = _seg_scatter(64)(x[:, :64], srcp, dstp, wp)
    xb = x[:, 64:] + 0.0 * agg1a[0, 0, 0]
    agg1b = _seg_scatter(64)(xb, srcp, dstp, wp)
    h1 = _tc(_b1_body, (_N, 8), agg1a, agg1b, x, W_rel1,
             b_rel1.reshape(1, -1), W_root1)
    agg2 = _seg_scatter(8)(h1, srcp, dstp, wp)
    h2 = _tc(_bi_body, (_N, 16), agg2, h1, W_rel2, b_rel2.reshape(1, -1),
             W_root2)
    agg3 = _seg_scatter(16)(h2, srcp, dstp, wp)
    h3 = _tc(_bi_body, (_N, 32), agg3, h2, W_rel3, b_rel3.reshape(1, -1),
             W_root3)
    agg4 = _seg_scatter(32)(h3, srcp, dstp, wp)
    h4 = _tc(_bi_body, (_N, 64), agg4, h3, W_rel4, b_rel4.reshape(1, -1),
             W_root4)
    agg5 = _seg_scatter(64)(h4, srcp, dstp, wp)
    out = _tc(_fin_body, (_G, _C), agg5, h4, W_rel5, b_rel5.reshape(1, -1),
              W_root5, batch2, W_lin, b_lin.reshape(1, -1))
    return out


# speedup vs baseline: 1.0597x; 1.0597x over previous
import dataclasses
import functools

import jax
import jax.numpy as jnp
from jax import lax
from jax.experimental import pallas as pl
from jax.experimental.pallas import tpu as pltpu
from jax.experimental.pallas import tpu_sc as plsc

_N = 10000
_G = 64
_E = 320000
_C = 2
_NCORES = 2
_NSUB = 16
_NW = _NCORES * _NSUB
_BLK = 128
_NBLK = 84
_EW = _BLK * _NBLK
_EPAD = _EW * _NW
_NPAD = 10240
_ROWS_OUT = _NPAD // _NSUB


def _sc_compiler_params():
    cp = pltpu.CompilerParams()
    fields = pltpu.CompilerParams.__dataclass_fields__
    if "needs_layout_passes" in fields:
        cp = dataclasses.replace(cp, needs_layout_passes=False)
    if "use_tc_tiling_on_sc" in fields:
        cp = dataclasses.replace(cp, use_tc_tiling_on_sc=False)
    return cp


@functools.lru_cache(maxsize=None)
def _seg_scatter(d: int):
    ld = d.bit_length() - 1
    mesh = plsc.VectorSubcoreMesh(core_axis_name="c", subcore_axis_name="s")
    sb = 2 if d >= 64 else 4
    nsup = _NBLK // sb
    se = sb * _BLK

    @functools.partial(
        pl.kernel,
        out_type=jax.ShapeDtypeStruct((_NCORES, _NPAD, d), jnp.float32),
        mesh=mesh,
        compiler_params=_sc_compiler_params(),
        scratch_types=[
            pltpu.VMEM((_EW,), jnp.int32),
            pltpu.VMEM((_EW,), jnp.int32),
            pltpu.VMEM((_EW,), jnp.float32),
            pltpu.VMEM((se, d), jnp.float32),
            pltpu.VMEM((se, d), jnp.float32),
            pltpu.VMEM((se, d), jnp.float32),
            pltpu.VMEM_SHARED((_NPAD, d), jnp.float32),
            pltpu.SemaphoreType.DMA,
            pltpu.SemaphoreType.DMA,
            pltpu.SemaphoreType.DMA,
            pltpu.SemaphoreType.DMA,
            pltpu.SemaphoreType.DMA,
            pltpu.SemaphoreType.DMA,
        ],
    )
    def k(h_hbm, src_hbm, dst_hbm, w_hbm, out_hbm,
          src_v, dst_v, w_v, r0, r1, r2, agg_sh,
          g0, g1, g2, s0, s1, s2):
        c = lax.axis_index("c")
        s = lax.axis_index("s")
        wid = c * _NSUB + s
        iota = lax.iota(jnp.int32, 16)
        zeros = jnp.zeros((16,), jnp.float32)
        rows = [r0, r1, r2]
        gsem = [g0, g1, g2]
        ssem = [s0, s1, s2]

        def _idx(v, j):
            return v.at[pl.ds(j * se, se)]

        def g_issue(j, t):
            pltpu.async_copy(h_hbm.at[_idx(src_v, j)], rows[t], gsem[t])

        def g_wait(j, t):
            pltpu.make_async_copy(h_hbm.at[_idx(src_v, j)], rows[t],
                                  gsem[t]).wait()

        def s_issue(j, t):
            pltpu.async_copy(rows[t], agg_sh.at[_idx(dst_v, j)],
                             ssem[t], add=True)

        def s_wait(j, t):
            pltpu.make_async_copy(rows[t], agg_sh.at[_idx(dst_v, j)],
                                  ssem[t]).wait()

        def mul(t, j):
            r = rows[t]
            base = j * se
            if d >= 16:
                @pl.loop(0, se, step=4)
                def _(e0):
                    for u in range(4):
                        e = e0 + u
                        wv = plsc.load_gather(
                            w_v, [lax.full((16,), base + e, jnp.int32)])
                        for kk in range(d // 16):
                            rv = r[e, pl.ds(kk * 16, 16)]
                            r[e, pl.ds(kk * 16, 16)] = rv * wv
            else:
                base_splat = lax.full((16,), base, jnp.int32)
                @pl.loop(0, se * d, step=64)
                def _(p0):
                    for u in range(4):
                        v = p0 + u * 16 + iota
                        ee = v >> ld
                        cc = v & (d - 1)
                        wv = plsc.load_gather(w_v, [base_splat + ee])
                        rv = plsc.load_gather(r, [ee, cc])
                        plsc.store_scatter(r, [ee, cc], rv * wv)

        pltpu.sync_copy(src_hbm.at[wid], src_v)
        pltpu.sync_copy(dst_hbm.at[wid], dst_v)
        pltpu.sync_copy(w_hbm.at[wid], w_v)
        g_issue(0, 0)
        g_issue(1, 1)

        @pl.loop(0, _BLK * d, step=16)
        def _(p):
            v = p + iota
            plsc.store_scatter(r2, [v >> ld, v & (d - 1)], zeros)

        @pl.loop(0, _ROWS_OUT // _BLK)
        def _(i):
            pltpu.sync_copy(r2.at[pl.ds(0, _BLK)],
                            agg_sh.at[pl.ds(s * _ROWS_OUT + i * _BLK, _BLK)])

        plsc.subcore_barrier()

        @pl.loop(0, nsup, step=3)
        def _(jj):
            for t in range(3):
                j = jj + t
                tn = (t + 2) % 3
                g_wait(j, t)
                mul(t, j)
                s_issue(j, t)

                @pl.when(j + 2 < nsup)
                def _():
                    if t == 0:
                        @pl.when(jj >= 1)
                        def _():
                            s_wait(j - 1, tn)
                    else:
                        s_wait(j - 1, tn)
                    g_issue(j + 2, tn)

        for t in range(3):
            s_wait(nsup - 3 + t, t)
        plsc.subcore_barrier()
        pltpu.sync_copy(agg_sh.at[pl.ds(s * _ROWS_OUT, _ROWS_OUT)],
                        out_hbm.at[c, pl.ds(s * _ROWS_OUT, _ROWS_OUT)])

    return k


def _dot(a, b):
    return jnp.dot(a, b, preferred_element_type=jnp.float32)


def _bi_body(a_ref, h_ref, wr_ref, br_ref, wt_ref, o_ref):
    agg = a_ref[0, :_N, :] + a_ref[1, :_N, :]
    o_ref[...] = (_dot(agg, wr_ref[...]) + br_ref[...]
                  + _dot(h_ref[...], wt_ref[...]))


def _b1_body(a0_ref, a1_ref, h_ref, wr_ref, br_ref, wt_ref, o_ref):
    agg = jnp.concatenate(
        [a0_ref[0, :_N, :] + a0_ref[1, :_N, :],
         a1_ref[0, :_N, :] + a1_ref[1, :_N, :]], axis=1)
    o_ref[...] = (_dot(agg, wr_ref[...]) + br_ref[...]
                  + _dot(h_ref[...], wt_ref[...]))


def _fin_body(a_ref, h_ref, wr_ref, br_ref, wt_ref, batch_ref,
              wl_ref, bl_ref, o_ref):
    agg = a_ref[0, :_N, :] + a_ref[1, :_N, :]
    h5 = (_dot(agg, wr_ref[...]) + br_ref[...]
          + _dot(h_ref[...], wt_ref[...]))
    sel = (batch_ref[...] == lax.broadcasted_iota(jnp.int32, (_G, _N), 0))
    pooled = jnp.dot(sel.astype(jnp.float32), h5,
                     preferred_element_type=jnp.float32,
                     precision=lax.Precision.HIGHEST)
    o_ref[...] = _dot(pooled, wl_ref[...]) + bl_ref[...]


def _tc(body, out_shape, *args):
    return pl.pallas_call(
        body, out_shape=jax.ShapeDtypeStruct(out_shape, jnp.float32))(*args)


def kernel(x, edge_index, edge_attr, batch,
           W_rel1, b_rel1, W_root1,
           W_rel2, b_rel2, W_root2,
           W_rel3, b_rel3, W_root3,
           W_rel4, b_rel4, W_root4,
           W_rel5, b_rel5, W_root5,
           W_lin, b_lin):
    f32 = jnp.float32
    src = edge_index[0]
    dst = edge_index[1]
    ppw = _EW - _E // _NW
    pad_idx = (jnp.arange(_NW, dtype=jnp.int32)[:, None] * ppw
               + jnp.arange(ppw, dtype=jnp.int32)[None, :]) % _N
    srcp = jnp.concatenate([src.reshape(_NW, _E // _NW), pad_idx],
                           axis=1).reshape(_NW, _EW)
    dstp = jnp.concatenate([dst.reshape(_NW, _E // _NW), pad_idx],
                           axis=1).reshape(_NW, _EW)
    wp = jnp.concatenate([edge_attr.reshape(_NW, _E // _NW),
                          jnp.zeros((_NW, ppw), f32)],
                         axis=1).reshape(_NW, _EW)
    batch2 = batch.reshape(1, _N)

    agg1a = _seg_scatter(64)(x[:, :64], srcp, dstp, wp)
    xb = x[:, 64:] + 0.0 * agg1a[0, 0, 0]
    agg1b = _seg_scatter(64)(xb, srcp, dstp, wp)
    h1 = _tc(_b1_body, (_N, 8), agg1a, agg1b, x, W_rel1,
             b_rel1.reshape(1, -1), W_root1)
    agg2 = _seg_scatter(8)(h1, srcp, dstp, wp)
    h2 = _tc(_bi_body, (_N, 16), agg2, h1, W_rel2, b_rel2.reshape(1, -1),
             W_root2)
    agg3 = _seg_scatter(16)(h2, srcp, dstp, wp)
    h3 = _tc(_bi_body, (_N, 32), agg3, h2, W_rel3, b_rel3.reshape(1, -1),
             W_root3)
    agg4 = _seg_scatter(32)(h3, srcp, dstp, wp)
    h4 = _tc(_bi_body, (_N, 64), agg4, h3, W_rel4, b_rel4.reshape(1, -1),
             W_root4)
    agg5 = _seg_scatter(64)(h4, srcp, dstp, wp)
    out = _tc(_fin_body, (_G, _C), agg5, h4, W_rel5, b_rel5.reshape(1, -1),
              W_root5, batch2, W_lin, b_lin.reshape(1, -1))
    return out


# speedup vs baseline: 1.0828x; 1.0218x over previous
import dataclasses
import functools

import jax
import jax.numpy as jnp
from jax import lax
from jax.experimental import pallas as pl
from jax.experimental.pallas import tpu as pltpu
from jax.experimental.pallas import tpu_sc as plsc

_N = 10000
_G = 64
_E = 320000
_C = 2
_NCORES = 2
_NSUB = 16
_NW = _NCORES * _NSUB
_BLK = 128
_NBLK = 84
_EW = _BLK * _NBLK
_EPAD = _EW * _NW
_NPAD = 10240
_ROWS_OUT = _NPAD // _NSUB


def _sc_compiler_params():
    cp = pltpu.CompilerParams()
    fields = pltpu.CompilerParams.__dataclass_fields__
    if "needs_layout_passes" in fields:
        cp = dataclasses.replace(cp, needs_layout_passes=False)
    if "use_tc_tiling_on_sc" in fields:
        cp = dataclasses.replace(cp, use_tc_tiling_on_sc=False)
    return cp


@functools.lru_cache(maxsize=None)
def _seg_scatter(d: int):
    ld = d.bit_length() - 1
    mesh = plsc.VectorSubcoreMesh(core_axis_name="c", subcore_axis_name="s")
    sb = 2 if d >= 64 else 4
    nsup = _NBLK // sb
    se = sb * _BLK

    @functools.partial(
        pl.kernel,
        out_type=jax.ShapeDtypeStruct((_NCORES, _NPAD, d), jnp.float32),
        mesh=mesh,
        compiler_params=_sc_compiler_params(),
        scratch_types=[
            pltpu.VMEM((_EW,), jnp.int32),
            pltpu.VMEM((_EW,), jnp.int32),
            pltpu.VMEM((_EW,), jnp.float32),
            pltpu.VMEM((se, d), jnp.float32),
            pltpu.VMEM((se, d), jnp.float32),
            pltpu.VMEM((se, d), jnp.float32),
            pltpu.VMEM_SHARED((_NPAD, d), jnp.float32),
            pltpu.SemaphoreType.DMA,
            pltpu.SemaphoreType.DMA,
            pltpu.SemaphoreType.DMA,
            pltpu.SemaphoreType.DMA,
            pltpu.SemaphoreType.DMA,
            pltpu.SemaphoreType.DMA,
        ],
    )
    def k(h_hbm, src_hbm, dst_hbm, w_hbm, out_hbm,
          src_v, dst_v, w_v, r0, r1, r2, agg_sh,
          g0, g1, g2, s0, s1, s2):
        c = lax.axis_index("c")
        s = lax.axis_index("s")
        wid = c * _NSUB + s
        iota = lax.iota(jnp.int32, 16)
        zeros = jnp.zeros((16,), jnp.float32)
        rows = [r0, r1, r2]
        gsem = [g0, g1, g2]
        ssem = [s0, s1, s2]

        def _idx(v, j):
            return v.at[pl.ds(j * se, se)]

        def g_issue(j, t):
            pltpu.async_copy(h_hbm.at[_idx(src_v, j)], rows[t], gsem[t])

        def g_wait(j, t):
            pltpu.make_async_copy(h_hbm.at[_idx(src_v, j)], rows[t],
                                  gsem[t]).wait()

        def s_issue(j, t):
            pltpu.async_copy(rows[t], agg_sh.at[_idx(dst_v, j)],
                             ssem[t], add=True)

        def s_wait(j, t):
            pltpu.make_async_copy(rows[t], agg_sh.at[_idx(dst_v, j)],
                                  ssem[t]).wait()

        def mul(t, j):
            r = rows[t]
            base = j * se
            if d >= 16:
                unr = 8 if d == 16 else 4
                @pl.loop(0, se, step=unr)
                def _(e0):
                    for u in range(unr):
                        e = e0 + u
                        wv = plsc.load_gather(
                            w_v, [lax.full((16,), base + e, jnp.int32)])
                        for kk in range(d // 16):
                            rv = r[e, pl.ds(kk * 16, 16)]
                            r[e, pl.ds(kk * 16, 16)] = rv * wv
            else:
                base_splat = lax.full((16,), base, jnp.int32)
                @pl.loop(0, se * d, step=128)
                def _(p0):
                    for u in range(8):
                        v = p0 + u * 16 + iota
                        ee = v >> ld
                        cc = v & (d - 1)
                        wv = plsc.load_gather(w_v, [base_splat + ee])
                        rv = plsc.load_gather(r, [ee, cc])
                        plsc.store_scatter(r, [ee, cc], rv * wv)

        cs = pltpu.async_copy(src_hbm.at[wid], src_v, s0)
        cd = pltpu.async_copy(dst_hbm.at[wid], dst_v, s1)
        cw = pltpu.async_copy(w_hbm.at[wid], w_v, s2)

        @pl.loop(0, _BLK * d, step=16)
        def _(p):
            v = p + iota
            plsc.store_scatter(r2, [v >> ld, v & (d - 1)], zeros)

        cs.wait()
        cd.wait()
        cw.wait()
        g_issue(0, 0)
        g_issue(1, 1)

        @pl.loop(0, _ROWS_OUT // _BLK)
        def _(i):
            pltpu.sync_copy(r2.at[pl.ds(0, _BLK)],
                            agg_sh.at[pl.ds(s * _ROWS_OUT + i * _BLK, _BLK)])

        plsc.subcore_barrier()

        @pl.loop(0, nsup, step=3)
        def _(jj):
            for t in range(3):
                j = jj + t
                tn = (t + 2) % 3
                g_wait(j, t)
                mul(t, j)
                s_issue(j, t)

                @pl.when(j + 2 < nsup)
                def _():
                    if t == 0:
                        @pl.when(jj >= 1)
                        def _():
                            s_wait(j - 1, tn)
                    else:
                        s_wait(j - 1, tn)
                    g_issue(j + 2, tn)

        for t in range(3):
            s_wait(nsup - 3 + t, t)
        plsc.subcore_barrier()
        pltpu.sync_copy(agg_sh.at[pl.ds(s * _ROWS_OUT, _ROWS_OUT)],
                        out_hbm.at[c, pl.ds(s * _ROWS_OUT, _ROWS_OUT)])

    return k


def _dot(a, b):
    return jnp.dot(a, b, preferred_element_type=jnp.float32)


def _bi_body(a_ref, h_ref, wr_ref, br_ref, wt_ref, o_ref):
    agg = a_ref[0, :_N, :] + a_ref[1, :_N, :]
    o_ref[...] = (_dot(agg, wr_ref[...]) + br_ref[...]
                  + _dot(h_ref[...], wt_ref[...]))


def _b1_body(a0_ref, a1_ref, h_ref, wr_ref, br_ref, wt_ref, o_ref):
    agg = jnp.concatenate(
        [a0_ref[0, :_N, :] + a0_ref[1, :_N, :],
         a1_ref[0, :_N, :] + a1_ref[1, :_N, :]], axis=1)
    o_ref[...] = (_dot(agg, wr_ref[...]) + br_ref[...]
                  + _dot(h_ref[...], wt_ref[...]))


def _fin_body(a_ref, h_ref, wr_ref, br_ref, wt_ref, batch_ref,
              wl_ref, bl_ref, o_ref):
    agg = a_ref[0, :_N, :] + a_ref[1, :_N, :]
    h5 = (_dot(agg, wr_ref[...]) + br_ref[...]
          + _dot(h_ref[...], wt_ref[...]))
    sel = (batch_ref[...] == lax.broadcasted_iota(jnp.int32, (_G, _N), 0))
    pooled = jnp.dot(sel.astype(jnp.float32), h5,
                     preferred_element_type=jnp.float32,
                     precision=lax.Precision.HIGHEST)
    o_ref[...] = _dot(pooled, wl_ref[...]) + bl_ref[...]


def _tc(body, out_shape, *args):
    return pl.pallas_call(
        body, out_shape=jax.ShapeDtypeStruct(out_shape, jnp.float32))(*args)


def kernel(x, edge_index, edge_attr, batch,
           W_rel1, b_rel1, W_root1,
           W_rel2, b_rel2, W_root2,
           W_rel3, b_rel3, W_root3,
           W_rel4, b_rel4, W_root4,
           W_rel5, b_rel5, W_root5,
           W_lin, b_lin):
    f32 = jnp.float32
    src = edge_index[0]
    dst = edge_index[1]
    ppw = _EW - _E // _NW
    pad_idx = (jnp.arange(_NW, dtype=jnp.int32)[:, None] * ppw
               + jnp.arange(ppw, dtype=jnp.int32)[None, :]) % _N
    srcp = jnp.concatenate([src.reshape(_NW, _E // _NW), pad_idx],
                           axis=1).reshape(_NW, _EW)
    dstp = jnp.concatenate([dst.reshape(_NW, _E // _NW), pad_idx],
                           axis=1).reshape(_NW, _EW)
    wp = jnp.concatenate([edge_attr.reshape(_NW, _E // _NW),
                          jnp.zeros((_NW, ppw), f32)],
                         axis=1).reshape(_NW, _EW)
    batch2 = batch.reshape(1, _N)

    agg1a = _seg_scatter(64)(x[:, :64], srcp, dstp, wp)
    xb = x[:, 64:] + 0.0 * agg1a[0, 0, 0]
    agg1b = _seg_scatter(64)(xb, srcp, dstp, wp)
    h1 = _tc(_b1_body, (_N, 8), agg1a, agg1b, x, W_rel1,
             b_rel1.reshape(1, -1), W_root1)
    agg2 = _seg_scatter(8)(h1, srcp, dstp, wp)
    h2 = _tc(_bi_body, (_N, 16), agg2, h1, W_rel2, b_rel2.reshape(1, -1),
             W_root2)
    agg3 = _seg_scatter(16)(h2, srcp, dstp, wp)
    h3 = _tc(_bi_body, (_N, 32), agg3, h2, W_rel3, b_rel3.reshape(1, -1),
             W_root3)
    agg4 = _seg_scatter(32)(h3, srcp, dstp, wp)
    h4 = _tc(_bi_body, (_N, 64), agg4, h3, W_rel4, b_rel4.reshape(1, -1),
             W_root4)
    agg5 = _seg_scatter(64)(h4, srcp, dstp, wp)
    out = _tc(_fin_body, (_G, _C), agg5, h4, W_rel5, b_rel5.reshape(1, -1),
              W_root5, batch2, W_lin, b_lin.reshape(1, -1))
    return out


# speedup vs baseline: 1.0875x; 1.0043x over previous
import dataclasses
import functools

import jax
import jax.numpy as jnp
from jax import lax
from jax.experimental import pallas as pl
from jax.experimental.pallas import tpu as pltpu
from jax.experimental.pallas import tpu_sc as plsc

_N = 10000
_G = 64
_E = 320000
_C = 2
_NCORES = 2
_NSUB = 16
_NW = _NCORES * _NSUB
_BLK = 128
_NBLK = 84
_EW = _BLK * _NBLK
_EPAD = _EW * _NW
_NPAD = 10240
_ROWS_OUT = _NPAD // _NSUB


def _sc_compiler_params():
    cp = pltpu.CompilerParams()
    fields = pltpu.CompilerParams.__dataclass_fields__
    if "needs_layout_passes" in fields:
        cp = dataclasses.replace(cp, needs_layout_passes=False)
    if "use_tc_tiling_on_sc" in fields:
        cp = dataclasses.replace(cp, use_tc_tiling_on_sc=False)
    return cp


@functools.lru_cache(maxsize=None)
def _seg_scatter(d: int, nh: int = 1):
    ld = d.bit_length() - 1
    mesh = plsc.VectorSubcoreMesh(core_axis_name="c", subcore_axis_name="s")
    sb = 2 if d >= 64 else 4
    nsup = _NBLK // sb
    se = sb * _BLK
    oshape = (nh, _NCORES, _NPAD, d) if nh > 1 else (_NCORES, _NPAD, d)

    @functools.partial(
        pl.kernel,
        out_type=jax.ShapeDtypeStruct(oshape, jnp.float32),
        mesh=mesh,
        compiler_params=_sc_compiler_params(),
        scratch_types=[
            pltpu.VMEM((_EW,), jnp.int32),
            pltpu.VMEM((_EW,), jnp.int32),
            pltpu.VMEM((_EW,), jnp.float32),
            pltpu.VMEM((se, d), jnp.float32),
            pltpu.VMEM((se, d), jnp.float32),
            pltpu.VMEM((se, d), jnp.float32),
            pltpu.VMEM_SHARED((_NPAD, d), jnp.float32),
            pltpu.SemaphoreType.DMA,
            pltpu.SemaphoreType.DMA,
            pltpu.SemaphoreType.DMA,
            pltpu.SemaphoreType.DMA,
            pltpu.SemaphoreType.DMA,
            pltpu.SemaphoreType.DMA,
        ],
    )
    def k(*refs):
        h_hbms = refs[:nh]
        src_hbm, dst_hbm, w_hbm, out_hbm = refs[nh:nh + 4]
        (src_v, dst_v, w_v, r0, r1, r2, agg_sh,
         g0, g1, g2, s0, s1, s2) = refs[nh + 4:]
        c = lax.axis_index("c")
        s = lax.axis_index("s")
        wid = c * _NSUB + s
        iota = lax.iota(jnp.int32, 16)
        zeros = jnp.zeros((16,), jnp.float32)
        rows = [r0, r1, r2]
        gsem = [g0, g1, g2]
        ssem = [s0, s1, s2]

        def _idx(v, j):
            return v.at[pl.ds(j * se, se)]

        def g_issue(h_hbm, j, t):
            pltpu.async_copy(h_hbm.at[_idx(src_v, j)], rows[t], gsem[t])

        def g_wait(h_hbm, j, t):
            pltpu.make_async_copy(h_hbm.at[_idx(src_v, j)], rows[t],
                                  gsem[t]).wait()

        def s_issue(j, t):
            pltpu.async_copy(rows[t], agg_sh.at[_idx(dst_v, j)],
                             ssem[t], add=True)

        def s_wait(j, t):
            pltpu.make_async_copy(rows[t], agg_sh.at[_idx(dst_v, j)],
                                  ssem[t]).wait()

        def mul(t, j):
            r = rows[t]
            base = j * se
            if d >= 16:
                unr = 8 if d == 16 else 4
                @pl.loop(0, se, step=unr)
                def _(e0):
                    for u in range(unr):
                        e = e0 + u
                        wv = plsc.load_gather(
                            w_v, [lax.full((16,), base + e, jnp.int32)])
                        for kk in range(d // 16):
                            rv = r[e, pl.ds(kk * 16, 16)]
                            r[e, pl.ds(kk * 16, 16)] = rv * wv
            else:
                base_splat = lax.full((16,), base, jnp.int32)
                @pl.loop(0, se * d, step=128)
                def _(p0):
                    for u in range(8):
                        v = p0 + u * 16 + iota
                        ee = v >> ld
                        cc = v & (d - 1)
                        wv = plsc.load_gather(w_v, [base_splat + ee])
                        rv = plsc.load_gather(r, [ee, cc])
                        plsc.store_scatter(r, [ee, cc], rv * wv)

        cs = pltpu.async_copy(src_hbm.at[wid], src_v, s0)
        cd = pltpu.async_copy(dst_hbm.at[wid], dst_v, s1)
        cw = pltpu.async_copy(w_hbm.at[wid], w_v, s2)

        for hi in range(nh):
            h_hbm = h_hbms[hi]

            @pl.loop(0, _BLK * d, step=16)
            def _(p):
                v = p + iota
                plsc.store_scatter(r2, [v >> ld, v & (d - 1)], zeros)

            if hi == 0:
                cs.wait()
                cd.wait()
                cw.wait()
            g_issue(h_hbm, 0, 0)
            g_issue(h_hbm, 1, 1)

            @pl.loop(0, _ROWS_OUT // _BLK)
            def _(i):
                pltpu.sync_copy(
                    r2.at[pl.ds(0, _BLK)],
                    agg_sh.at[pl.ds(s * _ROWS_OUT + i * _BLK, _BLK)])

            plsc.subcore_barrier()

            @pl.loop(0, nsup, step=3)
            def _(jj):
                for t in range(3):
                    j = jj + t
                    tn = (t + 2) % 3
                    g_wait(h_hbm, j, t)
                    mul(t, j)
                    s_issue(j, t)

                    @pl.when(j + 2 < nsup)
                    def _():
                        if t == 0:
                            @pl.when(jj >= 1)
                            def _():
                                s_wait(j - 1, tn)
                        else:
                            s_wait(j - 1, tn)
                        g_issue(h_hbm, j + 2, tn)

            for t in range(3):
                s_wait(nsup - 3 + t, t)
            plsc.subcore_barrier()
            dst_slice = (out_hbm.at[hi, c, pl.ds(s * _ROWS_OUT, _ROWS_OUT)]
                         if nh > 1 else
                         out_hbm.at[c, pl.ds(s * _ROWS_OUT, _ROWS_OUT)])
            pltpu.sync_copy(agg_sh.at[pl.ds(s * _ROWS_OUT, _ROWS_OUT)],
                            dst_slice)

    return k


def _dot(a, b):
    return jnp.dot(a, b, preferred_element_type=jnp.float32)


def _bi_body(a_ref, h_ref, wr_ref, br_ref, wt_ref, o_ref):
    agg = a_ref[0, :_N, :] + a_ref[1, :_N, :]
    o_ref[...] = (_dot(agg, wr_ref[...]) + br_ref[...]
                  + _dot(h_ref[...], wt_ref[...]))


def _b1_body(a_ref, h_ref, wr_ref, br_ref, wt_ref, o_ref):
    agg = jnp.concatenate(
        [a_ref[0, 0, :_N, :] + a_ref[0, 1, :_N, :],
         a_ref[1, 0, :_N, :] + a_ref[1, 1, :_N, :]], axis=1)
    o_ref[...] = (_dot(agg, wr_ref[...]) + br_ref[...]
                  + _dot(h_ref[...], wt_ref[...]))


def _fin_body(a_ref, h_ref, wr_ref, br_ref, wt_ref, batch_ref,
              wl_ref, bl_ref, o_ref):
    agg = a_ref[0, :_N, :] + a_ref[1, :_N, :]
    h5 = (_dot(agg, wr_ref[...]) + br_ref[...]
          + _dot(h_ref[...], wt_ref[...]))
    sel = (batch_ref[...] == lax.broadcasted_iota(jnp.int32, (_G, _N), 0))
    pooled = jnp.dot(sel.astype(jnp.float32), h5,
                     preferred_element_type=jnp.float32,
                     precision=lax.Precision.HIGHEST)
    o_ref[...] = _dot(pooled, wl_ref[...]) + bl_ref[...]


def _tc(body, out_shape, *args):
    return pl.pallas_call(
        body, out_shape=jax.ShapeDtypeStruct(out_shape, jnp.float32))(*args)


def kernel(x, edge_index, edge_attr, batch,
           W_rel1, b_rel1, W_root1,
           W_rel2, b_rel2, W_root2,
           W_rel3, b_rel3, W_root3,
           W_rel4, b_rel4, W_root4,
           W_rel5, b_rel5, W_root5,
           W_lin, b_lin):
    f32 = jnp.float32
    src = edge_index[0]
    dst = edge_index[1]
    ppw = _EW - _E // _NW
    pad_idx = (jnp.arange(_NW, dtype=jnp.int32)[:, None] * ppw
               + jnp.arange(ppw, dtype=jnp.int32)[None, :]) % _N
    srcp = jnp.concatenate([src.reshape(_NW, _E // _NW), pad_idx],
                           axis=1).reshape(_NW, _EW)
    dstp = jnp.concatenate([dst.reshape(_NW, _E // _NW), pad_idx],
                           axis=1).reshape(_NW, _EW)
    wp = jnp.concatenate([edge_attr.reshape(_NW, _E // _NW),
                          jnp.zeros((_NW, ppw), f32)],
                         axis=1).reshape(_NW, _EW)
    batch2 = batch.reshape(1, _N)

    agg1 = _seg_scatter(64, 2)(x[:, :64], x[:, 64:], srcp, dstp, wp)
    h1 = _tc(_b1_body, (_N, 8), agg1, x, W_rel1,
             b_rel1.reshape(1, -1), W_root1)
    agg2 = _seg_scatter(8)(h1, srcp, dstp, wp)
    h2 = _tc(_bi_body, (_N, 16), agg2, h1, W_rel2, b_rel2.reshape(1, -1),
             W_root2)
    agg3 = _seg_scatter(16)(h2, srcp, dstp, wp)
    h3 = _tc(_bi_body, (_N, 32), agg3, h2, W_rel3, b_rel3.reshape(1, -1),
             W_root3)
    agg4 = _seg_scatter(32)(h3, srcp, dstp, wp)
    h4 = _tc(_bi_body, (_N, 64), agg4, h3, W_rel4, b_rel4.reshape(1, -1),
             W_root4)
    agg5 = _seg_scatter(64)(h4, srcp, dstp, wp)
    out = _tc(_fin_body, (_G, _C), agg5, h4, W_rel5, b_rel5.reshape(1, -1),
              W_root5, batch2, W_lin, b_lin.reshape(1, -1))
    return out
